# Initial kernel scaffold; baseline (speedup 1.0000x reference)
import jax
import jax.numpy as jnp
from jax.experimental import pallas as pl


def kernel(x, cantor_coords, Wqkv, bqkv, Wout, bout):
    raise NotImplementedError("write your pallas kernel here")


# speedup vs baseline: 9.0370x; 9.0370x over previous
import functools
import math

import jax
import jax.numpy as jnp
from jax import lax
from jax.experimental import pallas as pl
from jax.experimental.pallas import tpu as pltpu
from jax.experimental.pallas import tpu_sc as plsc

S = 2048
D = 1024
H = 16
HD = 64
KN = 32
QT = 128
NQT = S // QT
RB = 256
SCALE = 1.0 / math.sqrt(HD)
NEG = -1e30


def _rank_body(c_col_ref, c_row_ref, rank_ref):
    i0 = pl.program_id(0) * RB
    ci = c_col_ref[...]
    cj = c_row_ref[...]
    ii = i0 + lax.broadcasted_iota(jnp.int32, (RB, 1), 0)
    jj = lax.broadcasted_iota(jnp.int32, (1, S), 1)
    less = (cj < ci) | ((cj == ci) & (jj < ii))
    rank_ref[...] = jnp.sum(less.astype(jnp.int32), axis=1, keepdims=True)


def _ranks(c_col, c_row):
    return pl.pallas_call(
        _rank_body,
        grid=(S // RB,),
        in_specs=[
            pl.BlockSpec((RB, 1), lambda i: (i, 0)),
            pl.BlockSpec((1, S), lambda i: (0, 0)),
        ],
        out_specs=pl.BlockSpec((RB, 1), lambda i: (i, 0)),
        out_shape=jax.ShapeDtypeStruct((S, 1), jnp.int32),
    )(c_col, c_row)


def _invert_body(rank_row_ref, c_row_ref, sidx_ref, cs_ref):
    r0 = pl.program_id(0) * RB
    ranks = rank_row_ref[...]
    c = c_row_ref[...]
    rr = r0 + lax.broadcasted_iota(jnp.int32, (RB, 1), 0)
    match = ranks == rr
    jj = lax.broadcasted_iota(jnp.int32, (1, S), 1)
    sidx_ref[...] = jnp.sum(jnp.where(match, jj, 0), axis=1, keepdims=True)
    cs_ref[...] = jnp.sum(jnp.where(match, c, 0.0), axis=1, keepdims=True)


def _invert(rank_row, c_row):
    return pl.pallas_call(
        _invert_body,
        grid=(S // RB,),
        in_specs=[
            pl.BlockSpec((1, S), lambda i: (0, 0)),
            pl.BlockSpec((1, S), lambda i: (0, 0)),
        ],
        out_specs=[
            pl.BlockSpec((RB, 1), lambda i: (i, 0)),
            pl.BlockSpec((RB, 1), lambda i: (i, 0)),
        ],
        out_shape=[
            jax.ShapeDtypeStruct((S, 1), jnp.int32),
            jax.ShapeDtypeStruct((S, 1), jnp.float32),
        ],
    )(rank_row, c_row)


def _window_body(csp_ref, csm_ref, csn_ref, l_ref):
    qt = pl.program_id(0)
    cs3 = jnp.concatenate(
        [csp_ref[0], csm_ref[0], csn_ref[0]], axis=1)
    cq = cs3[:, QT:2 * QT]
    r = qt * QT + lax.broadcasted_iota(jnp.int32, (1, QT), 1)
    best_cost = jnp.full((1, QT), jnp.inf, jnp.float32)
    best_w = jnp.zeros((1, QT), jnp.int32)
    for t in range(KN):
        lo = cs3[:, QT - t:2 * QT - t]
        hi = cs3[:, QT - t + KN - 1:2 * QT - t + KN - 1]
        cost = jnp.maximum(cq - lo, hi - cq)
        w = r - t
        valid = (w >= 0) & (w <= S - KN)
        cost = jnp.where(valid, cost, jnp.inf)
        upd = cost < best_cost
        best_cost = jnp.where(upd, cost, best_cost)
        best_w = jnp.where(upd, w, best_w)
    l_ref[0] = best_w


def _windows(cs_row3):
    return pl.pallas_call(
        _window_body,
        grid=(NQT,),
        in_specs=[
            pl.BlockSpec((1, 1, QT), lambda i: (jnp.maximum(i - 1, 0), 0, 0)),
            pl.BlockSpec((1, 1, QT), lambda i: (i, 0, 0)),
            pl.BlockSpec((1, 1, QT), lambda i: (jnp.minimum(i + 1, NQT - 1), 0, 0)),
        ],
        out_specs=pl.BlockSpec((1, 1, QT), lambda i: (i, 0, 0)),
        out_shape=jax.ShapeDtypeStruct((NQT, 1, QT), jnp.int32),
    )(cs_row3, cs_row3, cs_row3)


def _attn_body(l_ref, q_ref, kp_ref, km_ref, kn_ref, vp_ref, vm_ref, vn_ref,
               o_ref):
    qt = pl.program_id(0)
    k3 = jnp.concatenate([kp_ref[...], km_ref[...], kn_ref[...]], axis=0)
    v3 = jnp.concatenate([vp_ref[...], vm_ref[...], vn_ref[...]], axis=0)
    g = (qt - 1) * QT + lax.broadcasted_iota(jnp.int32, (3 * QT, 1), 0)
    lrow = l_ref[0]
    mask = (g >= lrow) & (g < lrow + KN)
    for hh in range(2):
        q = q_ref[:, hh * HD:(hh + 1) * HD]
        kh = k3[:, hh * HD:(hh + 1) * HD]
        vh = v3[:, hh * HD:(hh + 1) * HD]
        s = lax.dot_general(kh, q, (((1,), (1,)), ((), ())),
                            preferred_element_type=jnp.float32) * SCALE
        s = jnp.where(mask, s, NEG)
        m = jnp.max(s, axis=0, keepdims=True)
        p = jnp.exp(s - m)
        p = jnp.where(mask, p, 0.0)
        denom = jnp.sum(p, axis=0, keepdims=True)
        p = p / denom
        o = lax.dot_general(p, vh, (((0,), (0,)), ((), ())),
                            preferred_element_type=jnp.float32)
        o_ref[:, hh * HD:(hh + 1) * HD] = o


def _attention(l3, qkv):
    h2 = H // 2
    def kspec(col0):
        return [
            pl.BlockSpec((QT, 2 * HD), lambda i, h: (jnp.maximum(i - 1, 0), col0 + h)),
            pl.BlockSpec((QT, 2 * HD), lambda i, h: (i, col0 + h)),
            pl.BlockSpec((QT, 2 * HD), lambda i, h: (jnp.minimum(i + 1, NQT - 1), col0 + h)),
        ]
    return pl.pallas_call(
        _attn_body,
        grid=(NQT, h2),
        in_specs=[
            pl.BlockSpec((1, 1, QT), lambda i, h: (i, 0, 0)),
            pl.BlockSpec((QT, 2 * HD), lambda i, h: (i, h)),
            *kspec(h2),
            *kspec(2 * h2),
        ],
        out_specs=pl.BlockSpec((QT, 2 * HD), lambda i, h: (i, h)),
        out_shape=jax.ShapeDtypeStruct((S, D), jnp.float32),
    )(l3, qkv, qkv, qkv, qkv, qkv, qkv, qkv)


def _mm_body(x_ref, w_ref, b_ref, o_ref):
    o_ref[...] = (
        jnp.dot(x_ref[...], w_ref[...], preferred_element_type=jnp.float32)
        + b_ref[0:1, :])


def _matmul_bias(x, w, b8, bm=256, bn=256):
    m, k = x.shape
    n = w.shape[1]
    return pl.pallas_call(
        _mm_body,
        grid=(m // bm, n // bn),
        in_specs=[
            pl.BlockSpec((bm, k), lambda i, j: (i, 0)),
            pl.BlockSpec((k, bn), lambda i, j: (0, j)),
            pl.BlockSpec((8, bn), lambda i, j: (0, j)),
        ],
        out_specs=pl.BlockSpec((bm, bn), lambda i, j: (i, j)),
        out_shape=jax.ShapeDtypeStruct((m, n), jnp.float32),
    )(x, w, b8)


def _sc_gather(table, idx):
    ncol = table.shape[1]
    nw = 32
    bpw = S // nw
    mesh = plsc.VectorSubcoreMesh(core_axis_name="c", subcore_axis_name="s")

    @functools.partial(
        pl.kernel, mesh=mesh,
        out_type=jax.ShapeDtypeStruct((S, ncol), jnp.float32),
        scratch_types=[
            pltpu.VMEM((bpw,), jnp.int32),
            pltpu.VMEM((bpw, ncol), jnp.float32),
            pltpu.SemaphoreType.DMA,
        ],
    )
    def gk(table_hbm, idx_hbm, out_hbm, idx_v, rows_v, sem):
        wid = lax.axis_index("s") * 2 + lax.axis_index("c")
        base = wid * bpw
        pltpu.sync_copy(idx_hbm.at[pl.ds(base, bpw)], idx_v)
        pltpu.async_copy(table_hbm.at[idx_v], rows_v, sem).wait()
        pltpu.sync_copy(rows_v, out_hbm.at[pl.ds(base, bpw)])

    return gk(table, idx)


def kernel(x, cantor_coords, Wqkv, bqkv, Wout, bout):
    x2 = x.reshape(S, D)
    c_col = cantor_coords.reshape(S, 1)
    c_row = cantor_coords.reshape(1, S)

    rank_col = _ranks(c_col, c_row)
    sidx_col, cs_col = _invert(rank_col.reshape(1, S), c_row)
    l3 = _windows(cs_col.reshape(NQT, 1, QT))

    x_s = _sc_gather(x2, sidx_col.reshape(S))
    qkv = _matmul_bias(x_s, Wqkv, jnp.broadcast_to(bqkv, (8, 3 * D)))
    out_s = _attention(l3, qkv)
    y_s = _matmul_bias(out_s, Wout, jnp.broadcast_to(bout, (8, D)))
    y = _sc_gather(y_s, rank_col.reshape(S))
    return y.reshape(1, S, D)


# speedup vs baseline: 20.4580x; 2.2638x over previous
import functools
import math

import jax
import jax.numpy as jnp
from jax import lax
from jax.experimental import pallas as pl
from jax.experimental.pallas import tpu as pltpu
from jax.experimental.pallas import tpu_sc as plsc

S = 2048
D = 1024
H = 16
HD = 64
KN = 32
QT = 128
NQT = S // QT
RB = 256
SCALE = 1.0 / math.sqrt(HD)
NEG = -1e30


def _rank_body(c_col_ref, c_row_ref, rank_ref):
    i0 = pl.program_id(0) * RB
    ci = c_col_ref[...]
    cj = c_row_ref[...]
    ii = i0 + lax.broadcasted_iota(jnp.int32, (RB, 1), 0)
    jj = lax.broadcasted_iota(jnp.int32, (1, S), 1)
    less = (cj < ci) | ((cj == ci) & (jj < ii))
    rank_ref[...] = jnp.sum(less.astype(jnp.int32), axis=1, keepdims=True)


def _ranks(c_col, c_row):
    return pl.pallas_call(
        _rank_body,
        grid=(S // RB,),
        in_specs=[
            pl.BlockSpec((RB, 1), lambda i: (i, 0)),
            pl.BlockSpec((1, S), lambda i: (0, 0)),
        ],
        out_specs=pl.BlockSpec((RB, 1), lambda i: (i, 0)),
        out_shape=jax.ShapeDtypeStruct((S, 1), jnp.int32),
    )(c_col, c_row)


def _invert_body(rank_row_ref, c_row_ref, sidx_ref, cs_ref):
    r0 = pl.program_id(0) * RB
    ranks = rank_row_ref[...]
    c = c_row_ref[...]
    rr = r0 + lax.broadcasted_iota(jnp.int32, (RB, 1), 0)
    match = ranks == rr
    jj = lax.broadcasted_iota(jnp.int32, (1, S), 1)
    sidx_ref[...] = jnp.sum(jnp.where(match, jj, 0), axis=1, keepdims=True)
    cs_ref[...] = jnp.sum(jnp.where(match, c, 0.0), axis=1, keepdims=True)


def _invert(rank_row, c_row):
    return pl.pallas_call(
        _invert_body,
        grid=(S // RB,),
        in_specs=[
            pl.BlockSpec((1, S), lambda i: (0, 0)),
            pl.BlockSpec((1, S), lambda i: (0, 0)),
        ],
        out_specs=[
            pl.BlockSpec((RB, 1), lambda i: (i, 0)),
            pl.BlockSpec((RB, 1), lambda i: (i, 0)),
        ],
        out_shape=[
            jax.ShapeDtypeStruct((S, 1), jnp.int32),
            jax.ShapeDtypeStruct((S, 1), jnp.float32),
        ],
    )(rank_row, c_row)


def _attn_body(csp_ref, csm_ref, csn_ref, q_ref, kp_ref, km_ref, kn_ref,
               vp_ref, vm_ref, vn_ref, wo_ref, bo_ref, o_ref):
    qt = pl.program_id(0)
    cs3 = jnp.concatenate(
        [csp_ref[0], csm_ref[0], csn_ref[0]], axis=1)
    cq = cs3[:, QT:2 * QT]
    r = qt * QT + lax.broadcasted_iota(jnp.int32, (1, QT), 1)
    best_cost = jnp.full((1, QT), jnp.inf, jnp.float32)
    best_w = jnp.zeros((1, QT), jnp.int32)
    for t in range(KN):
        lo = cs3[:, QT - t:2 * QT - t]
        hi = cs3[:, QT - t + KN - 1:2 * QT - t + KN - 1]
        cost = jnp.maximum(cq - lo, hi - cq)
        w = r - t
        valid = (w >= 0) & (w <= S - KN)
        cost = jnp.where(valid, cost, jnp.inf)
        upd = cost < best_cost
        best_cost = jnp.where(upd, cost, best_cost)
        best_w = jnp.where(upd, w, best_w)

    k3 = jnp.concatenate([kp_ref[...], km_ref[...], kn_ref[...]], axis=0)
    v3 = jnp.concatenate([vp_ref[...], vm_ref[...], vn_ref[...]], axis=0)
    g = (qt - 1) * QT + lax.broadcasted_iota(jnp.int32, (3 * QT, 1), 0)
    mask = (g >= best_w) & (g < best_w + KN)
    q = q_ref[...] * SCALE
    outs = []
    for h in range(H):
        qh = q[:, h * HD:(h + 1) * HD]
        kh = k3[:, h * HD:(h + 1) * HD]
        vh = v3[:, h * HD:(h + 1) * HD]
        s = lax.dot_general(kh, qh, (((1,), (1,)), ((), ())),
                            preferred_element_type=jnp.float32)
        p = jnp.exp(jnp.where(mask, s, NEG))
        denom = jnp.sum(p, axis=0, keepdims=True)
        p = p * (1.0 / denom)
        outs.append(lax.dot_general(p, vh, (((0,), (0,)), ((), ())),
                                    preferred_element_type=jnp.float32))
    att = jnp.concatenate(outs, axis=1)
    o_ref[...] = (
        jnp.dot(att, wo_ref[...], preferred_element_type=jnp.float32)
        + bo_ref[0:1, :])


def _attention(cs3d, qkv, Wout, bout8):
    def band(col):
        return [
            pl.BlockSpec((QT, D), lambda i: (jnp.maximum(i - 1, 0), col)),
            pl.BlockSpec((QT, D), lambda i: (i, col)),
            pl.BlockSpec((QT, D), lambda i: (jnp.minimum(i + 1, NQT - 1), col)),
        ]
    return pl.pallas_call(
        _attn_body,
        grid=(NQT,),
        in_specs=[
            pl.BlockSpec((1, 1, QT), lambda i: (jnp.maximum(i - 1, 0), 0, 0)),
            pl.BlockSpec((1, 1, QT), lambda i: (i, 0, 0)),
            pl.BlockSpec((1, 1, QT), lambda i: (jnp.minimum(i + 1, NQT - 1), 0, 0)),
            pl.BlockSpec((QT, D), lambda i: (i, 0)),
            *band(1),
            *band(2),
            pl.BlockSpec((D, D), lambda i: (0, 0)),
            pl.BlockSpec((8, D), lambda i: (0, 0)),
        ],
        out_specs=pl.BlockSpec((QT, D), lambda i: (i, 0)),
        out_shape=jax.ShapeDtypeStruct((S, D), jnp.float32),
    )(cs3d, cs3d, cs3d, qkv, qkv, qkv, qkv, qkv, qkv, qkv, Wout, bout8)


def _mm_body(x_ref, w_ref, b_ref, o_ref):
    o_ref[...] = (
        jnp.dot(x_ref[...], w_ref[...], preferred_element_type=jnp.float32)
        + b_ref[0:1, :])


def _matmul_bias(x, w, b8, bn=256):
    m, k = x.shape
    n = w.shape[1]
    return pl.pallas_call(
        _mm_body,
        grid=(n // bn,),
        in_specs=[
            pl.BlockSpec((m, k), lambda j: (0, 0)),
            pl.BlockSpec((k, bn), lambda j: (0, j)),
            pl.BlockSpec((8, bn), lambda j: (0, j)),
        ],
        out_specs=pl.BlockSpec((m, bn), lambda j: (0, j)),
        out_shape=jax.ShapeDtypeStruct((m, n), jnp.float32),
    )(x, w, b8)


def _sc_gather(table, idx):
    ncol = table.shape[1]
    nw = 32
    bpw = S // nw
    mesh = plsc.VectorSubcoreMesh(core_axis_name="c", subcore_axis_name="s")

    @functools.partial(
        pl.kernel, mesh=mesh,
        out_type=jax.ShapeDtypeStruct((S, ncol), jnp.float32),
        scratch_types=[
            pltpu.VMEM((bpw,), jnp.int32),
            pltpu.VMEM((bpw, ncol), jnp.float32),
            pltpu.SemaphoreType.DMA,
        ],
    )
    def gk(table_hbm, idx_hbm, out_hbm, idx_v, rows_v, sem):
        wid = lax.axis_index("s") * 2 + lax.axis_index("c")
        base = wid * bpw
        pltpu.sync_copy(idx_hbm.at[pl.ds(base, bpw)], idx_v)
        pltpu.async_copy(table_hbm.at[idx_v], rows_v, sem).wait()
        pltpu.sync_copy(rows_v, out_hbm.at[pl.ds(base, bpw)])

    return gk(table, idx)


def kernel(x, cantor_coords, Wqkv, bqkv, Wout, bout):
    x2 = x.reshape(S, D)
    c_col = cantor_coords.reshape(S, 1)
    c_row = cantor_coords.reshape(1, S)

    rank_col = _ranks(c_col, c_row)
    sidx_col, cs_col = _invert(rank_col.reshape(1, S), c_row)

    x_s = _sc_gather(x2, sidx_col.reshape(S))
    qkv = _matmul_bias(x_s, Wqkv, jnp.broadcast_to(bqkv, (8, 3 * D)))
    y_s = _attention(cs_col.reshape(NQT, 1, QT), qkv, Wout,
                     jnp.broadcast_to(bout, (8, D)))
    y = _sc_gather(y_s, rank_col.reshape(S))
    return y.reshape(1, S, D)
